# baseline (speedup 1.0000x reference)
import functools

import jax
import jax.numpy as jnp
from jax import lax
from jax.experimental import pallas as pl
from jax.experimental.pallas import tpu as pltpu
from jax.experimental.pallas import tpu_sc as plsc

_NC = 2
_NS = 16
_LL = 16

_EDGE_CHUNK = 80


def _edge_pass(h, ea, edge_index):
    n, d = h.shape
    e = edge_index.shape[1]
    nw = _NC * _NS
    epw = e // nw
    K = _EDGE_CHUNK
    nchunks = epw // K
    rows_per_sub = (n // _NS) // 8 * 8
    rows_rem = n - rows_per_sub * _NS
    mesh = plsc.VectorSubcoreMesh(core_axis_name="c", subcore_axis_name="s")

    @functools.partial(
        pl.kernel,
        out_type=jax.ShapeDtypeStruct((_NC, n, d), jnp.float32),
        mesh=mesh,
        scratch_types=[
            pltpu.VMEM((2, 2, K), jnp.int32),
            pltpu.VMEM((2, K, d), jnp.float32),
            pltpu.VMEM((2, K, d), jnp.float32),
            pltpu.VMEM_SHARED((n, d), jnp.float32),
            pltpu.SemaphoreType.DMA((2,)),
            pltpu.SemaphoreType.DMA((2,)),
            pltpu.SemaphoreType.DMA((2,)),
        ],
    )
    def k(h_hbm, ea_hbm, src_hbm, dst_hbm, out_hbm, ibuf, gbuf, eabuf, aggr,
          sem_i, sem_g, sem_e):
        cid = lax.axis_index("c")
        sid = lax.axis_index("s")

        @pl.loop(0, K)
        def _(i):
            for j in range(d // _LL):
                gbuf[0, i, pl.ds(j * _LL, _LL)] = jnp.zeros((_LL,), jnp.float32)

        off = 0
        while off < rows_per_sub:
            sz = min(K, rows_per_sub - off)
            pltpu.sync_copy(
                gbuf.at[0, pl.ds(0, sz)],
                aggr.at[pl.ds(sid * rows_per_sub + off, sz)],
            )
            off += sz
        if rows_rem:
            @pl.when(sid == 0)
            def _():
                pltpu.sync_copy(
                    gbuf.at[0, pl.ds(0, rows_rem)],
                    aggr.at[pl.ds(rows_per_sub * _NS, rows_rem)],
                )
        plsc.subcore_barrier()

        base0 = (cid * _NS + sid) * epw

        def idx_copies(t, slot):
            return (
                pltpu.make_async_copy(
                    src_hbm.at[pl.ds(base0 + t * K, K)],
                    ibuf.at[slot, 0],
                    sem_i.at[slot],
                ),
                pltpu.make_async_copy(
                    dst_hbm.at[pl.ds(base0 + t * K, K)],
                    ibuf.at[slot, 1],
                    sem_i.at[slot],
                ),
            )

        def idx_start(t, slot):
            for c in idx_copies(t, slot):
                c.start()

        def idx_wait(t, slot):
            for c in idx_copies(t, slot):
                c.wait()

        def gather_copy(t, slot):
            del t
            return pltpu.make_async_copy(
                h_hbm.at[ibuf.at[slot, 0]],
                gbuf.at[slot],
                sem_g.at[slot],
            )

        def ea_copy(t, slot):
            return pltpu.make_async_copy(
                ea_hbm.at[pl.ds(base0 + t * K, K)],
                eabuf.at[slot],
                sem_e.at[slot],
            )

        idx_start(0, 0)
        idx_wait(0, 0)
        gather_copy(0, 0).start()
        ea_copy(0, 0).start()
        idx_start(1, 1)

        @pl.loop(0, nchunks)
        def _(t):
            slot = lax.rem(t, 2)
            nslot = 1 - slot

            @pl.when(t + 1 < nchunks)
            def _():
                idx_wait(t + 1, nslot)
                gather_copy(t + 1, nslot).start()
                ea_copy(t + 1, nslot).start()

            gather_copy(t, slot).wait()
            ea_copy(t, slot).wait()

            @pl.loop(0, K)
            def _(i):
                for j in range(d // _LL):
                    sl = pl.ds(j * _LL, _LL)
                    gbuf[slot, i, sl] = jnp.maximum(
                        gbuf[slot, i, sl] + eabuf[slot, i, sl], 0.0)

            pltpu.sync_copy(gbuf.at[slot], aggr.at[ibuf.at[slot, 1]], add=True)

            @pl.when(t + 2 < nchunks)
            def _():
                idx_start(t + 2, slot)

        plsc.subcore_barrier()

        off = 0
        while off < rows_per_sub:
            sz = min(K, rows_per_sub - off)
            row = sid * rows_per_sub + off
            pltpu.sync_copy(aggr.at[pl.ds(row, sz)],
                            out_hbm.at[cid, pl.ds(row, sz)])
            off += sz
        if rows_rem:
            @pl.when(sid == 0)
            def _():
                row = rows_per_sub * _NS
                pltpu.sync_copy(aggr.at[pl.ds(row, rows_rem)],
                                out_hbm.at[cid, pl.ds(row, rows_rem)])

    return k(h, ea, edge_index[0], edge_index[1])


def _linear(x, w, b, block_rows=None):
    m, kdim = x.shape
    nn = w.shape[1]
    if block_rows is None:
        block_rows = m
    b2 = b.reshape(1, nn)

    def body(x_ref, w_ref, b_ref, o_ref):
        o_ref[...] = (
            jnp.dot(x_ref[...], w_ref[...], preferred_element_type=jnp.float32)
            + b_ref[...]
        )

    return pl.pallas_call(
        body,
        grid=(m // block_rows,),
        in_specs=[
            pl.BlockSpec((block_rows, kdim), lambda i: (i, 0)),
            pl.BlockSpec((kdim, nn), lambda i: (0, 0)),
            pl.BlockSpec((1, nn), lambda i: (0, 0)),
        ],
        out_specs=pl.BlockSpec((block_rows, nn), lambda i: (i, 0)),
        out_shape=jax.ShapeDtypeStruct((m, nn), jnp.float32),
    )(x, w, b2)


def _gine_mlp(h, agg, lp):
    n, d = h.shape
    d2 = lp['W1'].shape[1]
    scale = (1.0 + lp['eps']).reshape(1, 1)

    def body(h_ref, a0_ref, a1_ref, s_ref, w1_ref, b1_ref, g1_ref, be1_ref,
             w2_ref, b2_ref, gn_ref, bn_ref, o_ref):
        z = s_ref[...] * h_ref[...] + a0_ref[...] + a1_ref[...]
        z = (
            jnp.dot(z, w1_ref[...], preferred_element_type=jnp.float32)
            + b1_ref[...]
        )
        mu = jnp.mean(z, axis=0, keepdims=True)
        zc = z - mu
        var = jnp.mean(zc * zc, axis=0, keepdims=True)
        z = zc * lax.rsqrt(var + 1e-5) * g1_ref[...] + be1_ref[...]
        z = jnp.maximum(z, 0.0)
        z = (
            jnp.dot(z, w2_ref[...], preferred_element_type=jnp.float32)
            + b2_ref[...]
        )
        mu2 = jnp.mean(z, axis=0, keepdims=True)
        zc2 = z - mu2
        var2 = jnp.mean(zc2 * zc2, axis=0, keepdims=True)
        z = zc2 * lax.rsqrt(var2 + 1e-5) * gn_ref[...] + bn_ref[...]
        o_ref[...] = jnp.maximum(z, 0.0)

    full = lambda shape: pl.BlockSpec(shape, lambda: (0,) * len(shape))
    return pl.pallas_call(
        body,
        in_specs=[
            full((n, d)), full((n, d)), full((n, d)), full((1, 1)),
            full((d, d2)), full((1, d2)), full((1, d2)), full((1, d2)),
            full((d2, d)), full((1, d)), full((1, d)), full((1, d)),
        ],
        out_specs=full((n, d)),
        out_shape=jax.ShapeDtypeStruct((n, d), jnp.float32),
    )(h, agg[0], agg[1], scale,
      lp['W1'], lp['b1'].reshape(1, d2), lp['g1'].reshape(1, d2),
      lp['be1'].reshape(1, d2),
      lp['W2'], lp['b2'].reshape(1, d), lp['gn'].reshape(1, d),
      lp['bn'].reshape(1, d))


def kernel(x, edge_index, edge_attr, params):
    h = _linear(x, params['W_atom'], params['b_atom'])
    ea = _linear(edge_attr, params['W_bond'], params['b_bond'], block_rows=8000)
    for lp in params['layers']:
        agg = _edge_pass(h, ea, edge_index)
        h = _gine_mlp(h, agg, lp)
    return _linear(h, params['W_out'], params['b_out'])


# speedup vs baseline: 2.6643x; 2.6643x over previous
import functools

import jax
import jax.numpy as jnp
from jax import lax
from jax.experimental import pallas as pl
from jax.experimental.pallas import tpu as pltpu
from jax.experimental.pallas import tpu_sc as plsc

_NC = 2
_NS = 16
_LL = 16

_EDGE_CHUNK = 80


def _edge_pass(h, ea, edge_index):
    n, d = h.shape
    e = edge_index.shape[1]
    nw = _NC * _NS
    epw = e // nw
    K = _EDGE_CHUNK
    nchunks = epw // K
    rows_per_sub = (n // _NS) // 8 * 8
    rows_rem = n - rows_per_sub * _NS
    mesh = plsc.VectorSubcoreMesh(core_axis_name="c", subcore_axis_name="s")

    @functools.partial(
        pl.kernel,
        out_type=jax.ShapeDtypeStruct((_NC, n, d), jnp.float32),
        mesh=mesh,
        scratch_types=[
            pltpu.VMEM((4, 2, K), jnp.int32),
            pltpu.VMEM((2, K, d), jnp.float32),
            pltpu.VMEM((2, K, d), jnp.float32),
            pltpu.VMEM_SHARED((n, d), jnp.float32),
            pltpu.SemaphoreType.DMA((4,)),
            pltpu.SemaphoreType.DMA((2,)),
            pltpu.SemaphoreType.DMA((2,)),
        ],
    )
    def k(h_hbm, ea_hbm, src_hbm, dst_hbm, out_hbm, ibuf, gbuf, eabuf, aggr,
          sem_i, sem_g, sem_e):
        cid = lax.axis_index("c")
        sid = lax.axis_index("s")

        @pl.loop(0, K)
        def _(i):
            for j in range(d // _LL):
                gbuf[0, i, pl.ds(j * _LL, _LL)] = jnp.zeros((_LL,), jnp.float32)

        off = 0
        while off < rows_per_sub:
            sz = min(K, rows_per_sub - off)
            pltpu.sync_copy(
                gbuf.at[0, pl.ds(0, sz)],
                aggr.at[pl.ds(sid * rows_per_sub + off, sz)],
            )
            off += sz
        if rows_rem:
            @pl.when(sid == 0)
            def _():
                pltpu.sync_copy(
                    gbuf.at[0, pl.ds(0, rows_rem)],
                    aggr.at[pl.ds(rows_per_sub * _NS, rows_rem)],
                )
        plsc.subcore_barrier()

        base0 = (cid * _NS + sid) * epw

        def idx_copies(t, slot):
            return (
                pltpu.make_async_copy(
                    src_hbm.at[pl.ds(base0 + t * K, K)],
                    ibuf.at[slot, 0],
                    sem_i.at[slot],
                ),
                pltpu.make_async_copy(
                    dst_hbm.at[pl.ds(base0 + t * K, K)],
                    ibuf.at[slot, 1],
                    sem_i.at[slot],
                ),
            )

        def idx_start(t, slot):
            for c in idx_copies(t, slot):
                c.start()

        def idx_wait(t, slot):
            for c in idx_copies(t, slot):
                c.wait()

        def gather_copy(t, slot):
            del t
            return pltpu.make_async_copy(
                h_hbm.at[ibuf.at[slot, 0]],
                gbuf.at[slot],
                sem_g.at[slot],
            )

        def ea_copy(t, slot):
            return pltpu.make_async_copy(
                ea_hbm.at[pl.ds(base0 + t * K, K)],
                eabuf.at[slot],
                sem_e.at[slot],
            )

        def compute_and_scatter(t, s2, s4):
            @pl.loop(0, K)
            def _(i):
                for j in range(d // _LL):
                    sl = pl.ds(j * _LL, _LL)
                    gbuf[s2, i, sl] = jnp.maximum(
                        gbuf[s2, i, sl] + eabuf[s2, i, sl], 0.0)

            pltpu.sync_copy(gbuf.at[s2], aggr.at[ibuf.at[s4, 1]], add=True)

        assert nchunks % 4 == 1 and nchunks >= 5
        idx_start(0, 0)
        idx_wait(0, 0)
        gather_copy(0, 0).start()
        ea_copy(0, 0).start()
        for p in (1, 2, 3):
            idx_start(p, p)

        @pl.loop(0, nchunks // 4)
        def _(j):
            t0 = j * 4
            for s in range(4):
                t = t0 + s
                s2, s4 = s % 2, s
                idx_wait(t + 1, (s + 1) % 4)
                gather_copy(t + 1, (s + 1) % 2).start()
                ea_copy(t + 1, (s + 1) % 2).start()
                gather_copy(t, s2).wait()
                ea_copy(t, s2).wait()
                compute_and_scatter(t, s2, s4)

                @pl.when(t + 4 < nchunks)
                def _():
                    idx_start(t + 4, s4)

        tl = nchunks - 1
        gather_copy(tl, 0).wait()
        ea_copy(tl, 0).wait()
        compute_and_scatter(tl, 0, 0)

        plsc.subcore_barrier()

        off = 0
        while off < rows_per_sub:
            sz = min(K, rows_per_sub - off)
            row = sid * rows_per_sub + off
            pltpu.sync_copy(aggr.at[pl.ds(row, sz)],
                            out_hbm.at[cid, pl.ds(row, sz)])
            off += sz
        if rows_rem:
            @pl.when(sid == 0)
            def _():
                row = rows_per_sub * _NS
                pltpu.sync_copy(aggr.at[pl.ds(row, rows_rem)],
                                out_hbm.at[cid, pl.ds(row, rows_rem)])

    return k(h, ea, edge_index[0], edge_index[1])


def _linear(x, w, b, block_rows=None):
    m, kdim = x.shape
    nn = w.shape[1]
    if block_rows is None:
        block_rows = m
    b2 = b.reshape(1, nn)

    def body(x_ref, w_ref, b_ref, o_ref):
        o_ref[...] = (
            jnp.dot(x_ref[...], w_ref[...], preferred_element_type=jnp.float32)
            + b_ref[...]
        )

    return pl.pallas_call(
        body,
        grid=(m // block_rows,),
        in_specs=[
            pl.BlockSpec((block_rows, kdim), lambda i: (i, 0)),
            pl.BlockSpec((kdim, nn), lambda i: (0, 0)),
            pl.BlockSpec((1, nn), lambda i: (0, 0)),
        ],
        out_specs=pl.BlockSpec((block_rows, nn), lambda i: (i, 0)),
        out_shape=jax.ShapeDtypeStruct((m, nn), jnp.float32),
    )(x, w, b2)


def _gine_mlp(h, agg, lp):
    n, d = h.shape
    d2 = lp['W1'].shape[1]
    scale = (1.0 + lp['eps']).reshape(1, 1)

    def body(h_ref, a0_ref, a1_ref, s_ref, w1_ref, b1_ref, g1_ref, be1_ref,
             w2_ref, b2_ref, gn_ref, bn_ref, o_ref):
        z = s_ref[...] * h_ref[...] + a0_ref[...] + a1_ref[...]
        z = (
            jnp.dot(z, w1_ref[...], preferred_element_type=jnp.float32)
            + b1_ref[...]
        )
        mu = jnp.mean(z, axis=0, keepdims=True)
        zc = z - mu
        var = jnp.mean(zc * zc, axis=0, keepdims=True)
        z = zc * lax.rsqrt(var + 1e-5) * g1_ref[...] + be1_ref[...]
        z = jnp.maximum(z, 0.0)
        z = (
            jnp.dot(z, w2_ref[...], preferred_element_type=jnp.float32)
            + b2_ref[...]
        )
        mu2 = jnp.mean(z, axis=0, keepdims=True)
        zc2 = z - mu2
        var2 = jnp.mean(zc2 * zc2, axis=0, keepdims=True)
        z = zc2 * lax.rsqrt(var2 + 1e-5) * gn_ref[...] + bn_ref[...]
        o_ref[...] = jnp.maximum(z, 0.0)

    full = lambda shape: pl.BlockSpec(shape, lambda: (0,) * len(shape))
    return pl.pallas_call(
        body,
        in_specs=[
            full((n, d)), full((n, d)), full((n, d)), full((1, 1)),
            full((d, d2)), full((1, d2)), full((1, d2)), full((1, d2)),
            full((d2, d)), full((1, d)), full((1, d)), full((1, d)),
        ],
        out_specs=full((n, d)),
        out_shape=jax.ShapeDtypeStruct((n, d), jnp.float32),
    )(h, agg[0], agg[1], scale,
      lp['W1'], lp['b1'].reshape(1, d2), lp['g1'].reshape(1, d2),
      lp['be1'].reshape(1, d2),
      lp['W2'], lp['b2'].reshape(1, d), lp['gn'].reshape(1, d),
      lp['bn'].reshape(1, d))


def kernel(x, edge_index, edge_attr, params):
    h = _linear(x, params['W_atom'], params['b_atom'])
    ea = _linear(edge_attr, params['W_bond'], params['b_bond'], block_rows=8000)
    for lp in params['layers']:
        agg = _edge_pass(h, ea, edge_index)
        h = _gine_mlp(h, agg, lp)
    return _linear(h, params['W_out'], params['b_out'])


# speedup vs baseline: 2.6662x; 1.0007x over previous
import functools

import jax
import jax.numpy as jnp
from jax import lax
from jax.experimental import pallas as pl
from jax.experimental.pallas import tpu as pltpu
from jax.experimental.pallas import tpu_sc as plsc

_NC = 2
_NS = 16
_LL = 16

_EDGE_CHUNK = 80


def _edge_pass(h, ea, edge_index):
    n, d = h.shape
    e = edge_index.shape[1]
    nw = _NC * _NS
    epw = e // nw
    K = _EDGE_CHUNK
    nchunks = epw // K
    rows_per_sub = (n // _NS) // 8 * 8
    rows_rem = n - rows_per_sub * _NS
    mesh = plsc.VectorSubcoreMesh(core_axis_name="c", subcore_axis_name="s")

    @functools.partial(
        pl.kernel,
        out_type=jax.ShapeDtypeStruct((_NC, n, d), jnp.float32),
        mesh=mesh,
        scratch_types=[
            pltpu.VMEM((4, 2, K), jnp.int32),
            pltpu.VMEM((2, K, d), jnp.float32),
            pltpu.VMEM((2, K, d), jnp.float32),
            pltpu.VMEM_SHARED((n, d), jnp.float32),
            pltpu.SemaphoreType.DMA((4,)),
            pltpu.SemaphoreType.DMA((2,)),
            pltpu.SemaphoreType.DMA((2,)),
        ],
    )
    def k(h_hbm, ea_hbm, src_hbm, dst_hbm, out_hbm, ibuf, gbuf, eabuf, aggr,
          sem_i, sem_g, sem_e):
        cid = lax.axis_index("c")
        sid = lax.axis_index("s")

        @pl.loop(0, K)
        def _(i):
            for j in range(d // _LL):
                gbuf[0, i, pl.ds(j * _LL, _LL)] = jnp.zeros((_LL,), jnp.float32)

        off = 0
        while off < rows_per_sub:
            sz = min(K, rows_per_sub - off)
            pltpu.sync_copy(
                gbuf.at[0, pl.ds(0, sz)],
                aggr.at[pl.ds(sid * rows_per_sub + off, sz)],
            )
            off += sz
        if rows_rem:
            @pl.when(sid == 0)
            def _():
                pltpu.sync_copy(
                    gbuf.at[0, pl.ds(0, rows_rem)],
                    aggr.at[pl.ds(rows_per_sub * _NS, rows_rem)],
                )
        plsc.subcore_barrier()

        base0 = (cid * _NS + sid) * epw

        def idx_copies(t, slot):
            return (
                pltpu.make_async_copy(
                    src_hbm.at[pl.ds(base0 + t * K, K)],
                    ibuf.at[slot, 0],
                    sem_i.at[slot],
                ),
                pltpu.make_async_copy(
                    dst_hbm.at[pl.ds(base0 + t * K, K)],
                    ibuf.at[slot, 1],
                    sem_i.at[slot],
                ),
            )

        def idx_start(t, slot):
            for c in idx_copies(t, slot):
                c.start()

        def idx_wait(t, slot):
            for c in idx_copies(t, slot):
                c.wait()

        def gather_copy(slot, islot):
            return pltpu.make_async_copy(
                h_hbm.at[ibuf.at[islot, 0]],
                gbuf.at[slot],
                sem_g.at[slot],
            )

        def ea_copy(t, slot):
            return pltpu.make_async_copy(
                ea_hbm.at[pl.ds(base0 + t * K, K)],
                eabuf.at[slot],
                sem_e.at[slot],
            )

        def compute_and_scatter(t, s2, s4):
            @pl.loop(0, K)
            def _(i):
                for j in range(d // _LL):
                    sl = pl.ds(j * _LL, _LL)
                    gbuf[s2, i, sl] = jnp.maximum(
                        gbuf[s2, i, sl] + eabuf[s2, i, sl], 0.0)

            pltpu.sync_copy(gbuf.at[s2], aggr.at[ibuf.at[s4, 1]], add=True)

        assert nchunks % 4 == 1 and nchunks >= 5
        idx_start(0, 0)
        idx_wait(0, 0)
        gather_copy(0, 0).start()
        ea_copy(0, 0).start()
        for p in (1, 2, 3):
            idx_start(p, p)

        @pl.loop(0, nchunks // 4)
        def _(j):
            t0 = j * 4
            for s in range(4):
                t = t0 + s
                s2, s4 = s % 2, s
                idx_wait(t + 1, (s + 1) % 4)
                gather_copy((s + 1) % 2, (s + 1) % 4).start()
                ea_copy(t + 1, (s + 1) % 2).start()
                gather_copy(s2, s4).wait()
                ea_copy(t, s2).wait()
                compute_and_scatter(t, s2, s4)

                @pl.when(t + 4 < nchunks)
                def _():
                    idx_start(t + 4, s4)

        tl = nchunks - 1
        gather_copy(0, 0).wait()
        ea_copy(tl, 0).wait()
        compute_and_scatter(tl, 0, 0)

        plsc.subcore_barrier()

        off = 0
        while off < rows_per_sub:
            sz = min(K, rows_per_sub - off)
            row = sid * rows_per_sub + off
            pltpu.sync_copy(aggr.at[pl.ds(row, sz)],
                            out_hbm.at[cid, pl.ds(row, sz)])
            off += sz
        if rows_rem:
            @pl.when(sid == 0)
            def _():
                row = rows_per_sub * _NS
                pltpu.sync_copy(aggr.at[pl.ds(row, rows_rem)],
                                out_hbm.at[cid, pl.ds(row, rows_rem)])

    return k(h, ea, edge_index[0], edge_index[1])


def _linear(x, w, b, block_rows=None):
    m, kdim = x.shape
    nn = w.shape[1]
    if block_rows is None:
        block_rows = m
    b2 = b.reshape(1, nn)

    def body(x_ref, w_ref, b_ref, o_ref):
        o_ref[...] = (
            jnp.dot(x_ref[...], w_ref[...], preferred_element_type=jnp.float32)
            + b_ref[...]
        )

    return pl.pallas_call(
        body,
        grid=(m // block_rows,),
        in_specs=[
            pl.BlockSpec((block_rows, kdim), lambda i: (i, 0)),
            pl.BlockSpec((kdim, nn), lambda i: (0, 0)),
            pl.BlockSpec((1, nn), lambda i: (0, 0)),
        ],
        out_specs=pl.BlockSpec((block_rows, nn), lambda i: (i, 0)),
        out_shape=jax.ShapeDtypeStruct((m, nn), jnp.float32),
    )(x, w, b2)


def _gine_mlp(h, agg, lp):
    n, d = h.shape
    d2 = lp['W1'].shape[1]
    scale = (1.0 + lp['eps']).reshape(1, 1)

    def body(h_ref, a0_ref, a1_ref, s_ref, w1_ref, b1_ref, g1_ref, be1_ref,
             w2_ref, b2_ref, gn_ref, bn_ref, o_ref):
        z = s_ref[...] * h_ref[...] + a0_ref[...] + a1_ref[...]
        z = (
            jnp.dot(z, w1_ref[...], preferred_element_type=jnp.float32)
            + b1_ref[...]
        )
        mu = jnp.mean(z, axis=0, keepdims=True)
        zc = z - mu
        var = jnp.mean(zc * zc, axis=0, keepdims=True)
        z = zc * lax.rsqrt(var + 1e-5) * g1_ref[...] + be1_ref[...]
        z = jnp.maximum(z, 0.0)
        z = (
            jnp.dot(z, w2_ref[...], preferred_element_type=jnp.float32)
            + b2_ref[...]
        )
        mu2 = jnp.mean(z, axis=0, keepdims=True)
        zc2 = z - mu2
        var2 = jnp.mean(zc2 * zc2, axis=0, keepdims=True)
        z = zc2 * lax.rsqrt(var2 + 1e-5) * gn_ref[...] + bn_ref[...]
        o_ref[...] = jnp.maximum(z, 0.0)

    full = lambda shape: pl.BlockSpec(shape, lambda: (0,) * len(shape))
    return pl.pallas_call(
        body,
        in_specs=[
            full((n, d)), full((n, d)), full((n, d)), full((1, 1)),
            full((d, d2)), full((1, d2)), full((1, d2)), full((1, d2)),
            full((d2, d)), full((1, d)), full((1, d)), full((1, d)),
        ],
        out_specs=full((n, d)),
        out_shape=jax.ShapeDtypeStruct((n, d), jnp.float32),
    )(h, agg[0], agg[1], scale,
      lp['W1'], lp['b1'].reshape(1, d2), lp['g1'].reshape(1, d2),
      lp['be1'].reshape(1, d2),
      lp['W2'], lp['b2'].reshape(1, d), lp['gn'].reshape(1, d),
      lp['bn'].reshape(1, d))


def kernel(x, edge_index, edge_attr, params):
    h = _linear(x, params['W_atom'], params['b_atom'])
    ea = _linear(edge_attr, params['W_bond'], params['b_bond'], block_rows=8000)
    for lp in params['layers']:
        agg = _edge_pass(h, ea, edge_index)
        h = _gine_mlp(h, agg, lp)
    return _linear(h, params['W_out'], params['b_out'])
